# baseline (speedup 1.0000x reference)
import functools

import jax
import jax.numpy as jnp
import numpy as np
from jax.experimental import pallas as pl
from jax.experimental.pallas import tpu as pltpu


def _hamilton(weight):
    r, i, j, k = jnp.split(weight, 4, axis=1)
    r2 = jnp.concatenate([r, -i, -j, -k], axis=1)
    i2 = jnp.concatenate([i, r, -k, j], axis=1)
    j2 = jnp.concatenate([j, k, r, -i], axis=1)
    k2 = jnp.concatenate([k, -j, i, r], axis=1)
    ham = jnp.concatenate([r2, i2, j2, k2], axis=0)
    n = ham.shape[0]
    p = (np.arange(n) * 4) % n + (np.arange(n) * 4) // n
    pinv = np.argsort(p)
    return ham[pinv][:, pinv]


def _body(adj_ref, x_ref, ham_ref, gamma_ref, beta_ref, out_ref,
          support_s, acc_s, sum_s, sq_s, scale_s, shift_s, *, bi, ni, n, bc):
    i = pl.program_id(0)

    @pl.when(i == 0)
    def _():
        support_s[...] = jnp.dot(x_ref[...], ham_ref[...],
                                 preferred_element_type=jnp.float32)

    @pl.when(i < ni)
    def _():
        blk = jnp.dot(adj_ref[...], support_s[...],
                      preferred_element_type=jnp.float32)
        acc_s[pl.ds(i * bi, bi), :] = blk

        psum = jnp.sum(blk, axis=0, keepdims=True)
        psq = jnp.sum(blk * blk, axis=0, keepdims=True)

        @pl.when(i == 0)
        def _():
            sum_s[...] = psum
            sq_s[...] = psq

        @pl.when(i > 0)
        def _():
            sum_s[...] += psum
            sq_s[...] += psq

        @pl.when(i == ni - 1)
        def _():
            mean = sum_s[...] * (1.0 / n)
            var = sq_s[...] * (1.0 / n) - mean * mean
            scale = jax.lax.rsqrt(var + 1e-5) * gamma_ref[...]
            scale_s[...] = scale
            shift_s[...] = beta_ref[...] - mean * scale

    @pl.when(i >= ni)
    def _():
        c = i - ni
        out_ref[...] = jnp.tanh(acc_s[pl.ds(c * bc, bc), :] * scale_s[...]
                                + shift_s[...])


def _pick_block(n, cap):
    for b in range(min(n, cap), 7, -1):
        if n % b == 0 and b % 8 == 0:
            return b
    return n


@jax.jit
def kernel(input, adj, weight, gamma, beta):
    n, f = input.shape
    ham = _hamilton(weight)
    bi = _pick_block(n, 400)
    ni = n // bi
    bc = _pick_block(n, 2000)
    nt = n // bc

    gamma2 = gamma.reshape(1, f)
    beta2 = beta.reshape(1, f)

    out = pl.pallas_call(
        functools.partial(_body, bi=bi, ni=ni, n=n, bc=bc),
        grid=(ni + nt,),
        in_specs=[
            pl.BlockSpec((bi, n), lambda i, _ni=ni: (jnp.minimum(i, _ni - 1), 0)),
            pl.BlockSpec((n, f), lambda i: (0, 0)),
            pl.BlockSpec((f, f), lambda i: (0, 0)),
            pl.BlockSpec((1, f), lambda i: (0, 0)),
            pl.BlockSpec((1, f), lambda i: (0, 0)),
        ],
        out_specs=pl.BlockSpec((bc, f), lambda i, _ni=ni: (jnp.maximum(i - _ni, 0), 0)),
        out_shape=jax.ShapeDtypeStruct((n, f), jnp.float32),
        scratch_shapes=[
            pltpu.VMEM((n, f), jnp.float32),
            pltpu.VMEM((n, f), jnp.float32),
            pltpu.VMEM((1, f), jnp.float32),
            pltpu.VMEM((1, f), jnp.float32),
            pltpu.VMEM((1, f), jnp.float32),
            pltpu.VMEM((1, f), jnp.float32),
        ],
        compiler_params=pltpu.CompilerParams(
            dimension_semantics=("arbitrary",),
        ),
    )(adj, input, ham, gamma2, beta2)
    return out


# speedup vs baseline: 1.0037x; 1.0037x over previous
import functools

import jax
import jax.numpy as jnp
import numpy as np
from jax.experimental import pallas as pl
from jax.experimental.pallas import tpu as pltpu


def _hamilton(weight):
    r, i, j, k = jnp.split(weight, 4, axis=1)
    r2 = jnp.concatenate([r, -i, -j, -k], axis=1)
    i2 = jnp.concatenate([i, r, -k, j], axis=1)
    j2 = jnp.concatenate([j, k, r, -i], axis=1)
    k2 = jnp.concatenate([k, -j, i, r], axis=1)
    ham = jnp.concatenate([r2, i2, j2, k2], axis=0)
    n = ham.shape[0]
    p = (np.arange(n) * 4) % n + (np.arange(n) * 4) // n
    pinv = np.argsort(p)
    return ham[pinv][:, pinv]


def _body(adj_ref, x_ref, ham_ref, gamma_ref, beta_ref, out_ref,
          support_s, sum_s, sq_s, *, bi, ni, n):
    i = pl.program_id(0)

    @pl.when(i == 0)
    def _():
        support_s[...] = jnp.dot(x_ref[...], ham_ref[...],
                                 preferred_element_type=jnp.float32)

    blk = jnp.dot(adj_ref[...], support_s[...],
                  preferred_element_type=jnp.float32)
    out_ref[pl.ds(i * bi, bi), :] = blk

    psum = jnp.sum(blk, axis=0, keepdims=True)
    psq = jnp.sum(blk * blk, axis=0, keepdims=True)

    @pl.when(i == 0)
    def _():
        sum_s[...] = psum
        sq_s[...] = psq

    @pl.when(i > 0)
    def _():
        sum_s[...] += psum
        sq_s[...] += psq

    @pl.when(i == ni - 1)
    def _():
        mean = sum_s[...] * (1.0 / n)
        var = sq_s[...] * (1.0 / n) - mean * mean
        scale = jax.lax.rsqrt(var + 1e-5) * gamma_ref[...]
        shift = beta_ref[...] - mean * scale
        out_ref[...] = jnp.tanh(out_ref[...] * scale + shift)


def _pick_block(n, cap):
    for b in range(min(n, cap), 7, -1):
        if n % b == 0 and b % 8 == 0:
            return b
    return n


@jax.jit
def kernel(input, adj, weight, gamma, beta):
    n, f = input.shape
    ham = _hamilton(weight)
    bi = _pick_block(n, 400)
    ni = n // bi

    gamma2 = gamma.reshape(1, f)
    beta2 = beta.reshape(1, f)

    out = pl.pallas_call(
        functools.partial(_body, bi=bi, ni=ni, n=n),
        grid=(ni,),
        in_specs=[
            pl.BlockSpec((bi, n), lambda i: (i, 0)),
            pl.BlockSpec((n, f), lambda i: (0, 0)),
            pl.BlockSpec((f, f), lambda i: (0, 0)),
            pl.BlockSpec((1, f), lambda i: (0, 0)),
            pl.BlockSpec((1, f), lambda i: (0, 0)),
        ],
        out_specs=pl.BlockSpec((n, f), lambda i: (0, 0)),
        out_shape=jax.ShapeDtypeStruct((n, f), jnp.float32),
        scratch_shapes=[
            pltpu.VMEM((n, f), jnp.float32),
            pltpu.VMEM((1, f), jnp.float32),
            pltpu.VMEM((1, f), jnp.float32),
        ],
        compiler_params=pltpu.CompilerParams(
            dimension_semantics=("arbitrary",),
        ),
    )(adj, input, ham, gamma2, beta2)
    return out
